# baseline (speedup 1.0000x reference)
import functools

import jax
import jax.numpy as jnp
from jax import lax
from jax.experimental import pallas as pl
from jax.experimental.pallas import tpu as pltpu
from jax.experimental.pallas import tpu_sc as plsc

N_USERS = 100000
N_ITEMS = 1000
BATCH = 16384

_NC = 2
_NS = 16
_NW = _NC * _NS
_LANES = 16

_B_PER_W = BATCH // _NW
_N_STREAMS = 4
_IDX_PER_STREAM = _B_PER_W // _N_STREAMS


def _gather_body(users_hbm, items_hbm, flat_scores_hbm, out_hbm,
                 uv, tv, idx2, vals2, sem):
  wid = lax.axis_index("s") * _NC + lax.axis_index("c")
  base = wid * _B_PER_W

  pltpu.sync_copy(users_hbm.at[pl.ds(base, _B_PER_W)], uv)
  pltpu.sync_copy(items_hbm.at[pl.ds(base, _B_PER_W)], tv)

  for j in range(_N_STREAMS):
    for c in range(_IDX_PER_STREAM // _LANES):
      off = j * _IDX_PER_STREAM + c * _LANES
      u = uv[pl.ds(off, _LANES)]
      t = tv[pl.ds(off, _LANES)]
      idx2[j, pl.ds(c * _LANES, _LANES)] = u * N_ITEMS + t

  copies = [
      pltpu.async_copy(flat_scores_hbm.at[idx2.at[j]], vals2.at[j], sem)
      for j in range(_N_STREAMS)
  ]
  for c in copies:
    c.wait()

  for j in range(_N_STREAMS):
    pltpu.sync_copy(
        vals2.at[j],
        out_hbm.at[pl.ds(base + j * _IDX_PER_STREAM, _IDX_PER_STREAM)])


@jax.jit
def _sc_gather(users, items, flat_scores):
  mesh = plsc.VectorSubcoreMesh(core_axis_name="c", subcore_axis_name="s")
  return pl.kernel(
      _gather_body,
      out_type=jax.ShapeDtypeStruct((BATCH,), jnp.float32),
      mesh=mesh,
      scratch_types=[
          pltpu.VMEM((_B_PER_W,), jnp.int32),
          pltpu.VMEM((_B_PER_W,), jnp.int32),
          pltpu.VMEM((_N_STREAMS, _IDX_PER_STREAM), jnp.int32),
          pltpu.VMEM((_N_STREAMS, _IDX_PER_STREAM), jnp.float32),
          pltpu.SemaphoreType.DMA,
      ],
  )(users, items, flat_scores)


def kernel(users, items, scores):
  users = users.astype(jnp.int32)
  items = items.astype(jnp.int32)
  flat_scores = scores.reshape(N_USERS * N_ITEMS)
  return _sc_gather(users, items, flat_scores)


# speedup vs baseline: 80.9082x; 80.9082x over previous
import jax
import jax.numpy as jnp
from jax import lax
from jax.experimental import pallas as pl
from jax.experimental.pallas import tpu as pltpu
from jax.experimental.pallas import tpu_sc as plsc

N_USERS = 100000
N_ITEMS = 1000
BATCH = 16384

_NC = 2
_NS = 16
_NW = _NC * _NS
_L = 16
_B_PER_W = BATCH // _NW
_NGRP = _B_PER_W // _L


def _gather_body(users_hbm, items_hbm, tscores_hbm, out_hbm,
                 uv, tv, grp, vals, sem):
  wid = lax.axis_index("s") * _NC + lax.axis_index("c")
  base = wid * _B_PER_W
  pltpu.sync_copy(users_hbm.at[pl.ds(base, _B_PER_W)], uv)
  pltpu.sync_copy(items_hbm.at[pl.ds(base, _B_PER_W)], tv)

  def issue(g, _):
    u_vec = uv[pl.ds(g * _L, _L)]
    t_vec = tv[pl.ds(g * _L, _L)]
    for j in range(_L):
      u = u_vec[j]
      it = t_vec[j]
      u0 = (u // 8) * 8
      pltpu.async_copy(tscores_hbm.at[it, pl.ds(u0, 8)],
                       grp.at[pl.ds((g * _L + j) * 8, 8)], sem)
    return 0

  lax.fori_loop(0, _NGRP, issue, 0)

  pltpu.make_async_copy(out_hbm.at[pl.ds(0, _B_PER_W * 8)], grp, sem).wait()

  for g in range(_NGRP):
    rows = lax.iota(jnp.int32, _L) * 8 + g * _L * 8
    cols = uv[pl.ds(g * _L, _L)] & 7
    vals[pl.ds(g * _L, _L)] = plsc.load_gather(grp, [rows + cols])

  pltpu.sync_copy(vals, out_hbm.at[pl.ds(base, _B_PER_W)])


def kernel(users, items, scores):
  users = users.astype(jnp.int32)
  items = items.astype(jnp.int32)
  ts = scores.T
  mesh = plsc.VectorSubcoreMesh(core_axis_name="c", subcore_axis_name="s")
  return pl.kernel(
      _gather_body,
      out_type=jax.ShapeDtypeStruct((BATCH,), jnp.float32),
      mesh=mesh,
      scratch_types=[
          pltpu.VMEM((_B_PER_W,), jnp.int32),
          pltpu.VMEM((_B_PER_W,), jnp.int32),
          pltpu.VMEM((_B_PER_W * 8,), jnp.float32),
          pltpu.VMEM((_B_PER_W,), jnp.float32),
          pltpu.SemaphoreType.DMA,
      ],
      compiler_params=pltpu.CompilerParams(use_tc_tiling_on_sc=True,
                                           needs_layout_passes=False),
  )(users, items, ts)


# speedup vs baseline: 90.8261x; 1.1226x over previous
import jax
import jax.numpy as jnp
from jax import lax
from jax.experimental import pallas as pl
from jax.experimental.pallas import tpu as pltpu
from jax.experimental.pallas import tpu_sc as plsc

N_USERS = 100000
N_ITEMS = 1000
BATCH = 16384

_NC = 2
_NS = 16
_NW = _NC * _NS
_L = 16
_B_PER_W = BATCH // _NW
_NGRP = _B_PER_W // _L


def _gather_body(packed_hbm, tscores_hbm, out_hbm, pv, grp, vals, sem):
  wid = lax.axis_index("s") * _NC + lax.axis_index("c")
  base = wid * _B_PER_W
  pltpu.sync_copy(packed_hbm.at[pl.ds(base, _B_PER_W)], pv)

  def issue(g, _):
    p_vec = pv[pl.ds(g * _L, _L)]
    for j in range(_L):
      p = p_vec[j]
      it = p >> 17
      u0 = ((p & 131071) >> 3) * 8
      pltpu.async_copy(tscores_hbm.at[it, pl.ds(u0, 8)],
                       grp.at[pl.ds((g * _L + j) * 8, 8)], sem)
    return 0

  lax.fori_loop(0, _NGRP, issue, 0)

  pltpu.make_async_copy(out_hbm.at[pl.ds(0, _B_PER_W * 8)], grp, sem).wait()

  def extract(g, _):
    rows = lax.iota(jnp.int32, _L) * 8 + g * (_L * 8)
    lanes = pv[pl.ds(g * _L, _L)] & 7
    vals[pl.ds(g * _L, _L)] = plsc.load_gather(grp, [rows + lanes])
    return 0

  lax.fori_loop(0, _NGRP, extract, 0)

  pltpu.sync_copy(vals, out_hbm.at[pl.ds(base, _B_PER_W)])


def kernel(users, items, scores):
  packed = (items.astype(jnp.int32) << 17) | users.astype(jnp.int32)
  ts = scores.T
  mesh = plsc.VectorSubcoreMesh(core_axis_name="c", subcore_axis_name="s")
  return pl.kernel(
      _gather_body,
      out_type=jax.ShapeDtypeStruct((BATCH,), jnp.float32),
      mesh=mesh,
      scratch_types=[
          pltpu.VMEM((_B_PER_W,), jnp.int32),
          pltpu.VMEM((_B_PER_W * 8,), jnp.float32),
          pltpu.VMEM((_B_PER_W,), jnp.float32),
          pltpu.SemaphoreType.DMA,
      ],
      compiler_params=pltpu.CompilerParams(use_tc_tiling_on_sc=True,
                                           needs_layout_passes=False),
  )(packed, ts)
